# baseline (speedup 1.0000x reference)
import functools

import jax
import jax.numpy as jnp
from jax import lax
from jax.experimental import pallas as pl
from jax.experimental.pallas import tpu as pltpu
from jax.experimental.pallas import tpu_sc as plsc

NC = 2
NS = 16
NW = NC * NS
CHUNK = 128
GRP = 1
NBUF = 2
DEGW = 16


def _sc_aggregate_build(n_acc, half, nchunk, with_deg, gw=None):
    gw = half if gw is None else gw
    rpt = n_acc // NS

    out_type = [jax.ShapeDtypeStruct((NC * n_acc, half), jnp.float32)]
    scratch = [
        pltpu.VMEM((nchunk, CHUNK), jnp.int32),
        pltpu.VMEM((nchunk, CHUNK), jnp.int32),
        pltpu.VMEM((NBUF, CHUNK, gw), jnp.float32),
        pltpu.VMEM_SHARED((n_acc, half), jnp.float32),
    ] + [pltpu.SemaphoreType.DMA] * NBUF
    if with_deg:
        out_type.append(jax.ShapeDtypeStruct((NC * n_acc, DEGW), jnp.float32))
        scratch.append(pltpu.VMEM((CHUNK, DEGW), jnp.float32))
        scratch.append(pltpu.VMEM_SHARED((n_acc, DEGW), jnp.float32))

    mesh = plsc.VectorSubcoreMesh(core_axis_name="c", subcore_axis_name="s")

    def body(h_hbm, src_hbm, dst_hbm, zeros_hbm, zeros16_hbm, ones_hbm,
             *refs):
        if with_deg:
            (c_out, d_out, src_v, dst_v, rows_v, acc_s, *sems,
             ones_v, deg_s) = refs
        else:
            (c_out, src_v, dst_v, rows_v, acc_s, *sems) = refs
        gsem = sems
        cid = lax.axis_index("c")
        sid = lax.axis_index("s")
        wid = cid * NS + sid
        r0 = sid * rpt

        pltpu.sync_copy(zeros_hbm.at[pl.ds(r0, rpt)], acc_s.at[pl.ds(r0, rpt)])
        if with_deg:
            pltpu.sync_copy(zeros16_hbm.at[pl.ds(r0, rpt)],
                            deg_s.at[pl.ds(r0, rpt)])
            pltpu.sync_copy(ones_hbm, ones_v)

        pltpu.sync_copy(src_hbm.at[wid], src_v)
        pltpu.sync_copy(dst_hbm.at[wid], dst_v)
        plsc.subcore_barrier()

        ngroups = nchunk // GRP

        def sidx(g):
            return src_v.at[g]

        def didx(g):
            return dst_v.at[g]

        for b in range(NBUF):
            pltpu.async_copy(h_hbm.at[sidx(b)], rows_v.at[b], gsem[b])

        def step(s, carry):
            for b in range(NBUF):
                g = s * NBUF + b
                pltpu.make_async_copy(
                    h_hbm.at[sidx(g)], rows_v.at[b], gsem[b]).wait()

                @pl.when(g + NBUF < ngroups)
                def _():
                    pltpu.async_copy(
                        h_hbm.at[sidx(g + NBUF)], rows_v.at[b], gsem[b])
            return carry

        lax.fori_loop(0, ngroups // NBUF, step, 0)
        plsc.subcore_barrier()

        base = cid * n_acc + r0
        pltpu.sync_copy(acc_s.at[pl.ds(r0, rpt)], c_out.at[pl.ds(base, rpt)])
        if with_deg:
            pltpu.sync_copy(deg_s.at[pl.ds(r0, rpt)],
                            d_out.at[pl.ds(base, rpt)])

    return pl.kernel(body, out_type=out_type, scratch_types=scratch,
                     mesh=mesh,
                     compiler_params=pltpu.CompilerParams(
                         use_tc_tiling_on_sc=False))


def _tc_layer_body(act, h_ref, cl_ref, cr_ref, dg_ref, w_ref, b_ref, o_ref):
    c = jnp.concatenate([cl_ref[...], cr_ref[...]], axis=1)
    deg = dg_ref[:, 0:1]
    c = c / jnp.maximum(deg, 1.0)
    h = h_ref[...]
    dh = h.shape[1]
    bundle = (jnp.dot(h, w_ref[:dh, :], preferred_element_type=jnp.float32)
              + jnp.dot(c, w_ref[dh:, :], preferred_element_type=jnp.float32)
              + b_ref[...])
    nrm = jnp.maximum(
        jnp.sqrt(jnp.sum(bundle * bundle, axis=1, keepdims=True)), 1e-12)
    bundle = bundle / nrm
    if act:
        bundle = jnp.maximum(bundle, 0.0)
    o_ref[...] = bundle


def _tc_layer(h, cl, cr, dg, w, b, act, block_rows):
    n, d = h.shape
    half = d // 2
    grid = (n // block_rows,)
    row_spec = pl.BlockSpec((block_rows, d), lambda i: (i, 0))
    half_spec = pl.BlockSpec((block_rows, half), lambda i: (i, 0))
    deg_spec = pl.BlockSpec((block_rows, DEGW), lambda i: (i, 0))
    return pl.pallas_call(
        functools.partial(_tc_layer_body, act),
        grid=grid,
        in_specs=[
            row_spec, half_spec, half_spec, deg_spec,
            pl.BlockSpec((2 * d, d), lambda i: (0, 0)),
            pl.BlockSpec((1, d), lambda i: (0, 0)),
        ],
        out_specs=row_spec,
        out_shape=jax.ShapeDtypeStruct((n, d), jnp.float32),
    )(h, cl, cr, dg, w, b.reshape(1, d))


def kernel(x, edge_index, W0, b0, W1, b1, W2, b2):
    n, d = x.shape
    e = edge_index.shape[1]
    half = d // 2

    per = -(-e // NS)
    q = GRP * NBUF
    nchunk = -(-(-(-per // CHUNK)) // q) * q
    pt = nchunk * CHUNK
    src = edge_index[0]
    dst = edge_index[1]
    pad_total = NS * pt - e
    src_t = jnp.concatenate(
        [src, jnp.zeros((pad_total,), jnp.int32)]).reshape(NS, nchunk, CHUNK)
    dst_t = jnp.concatenate(
        [dst, jnp.full((pad_total,), n, jnp.int32)]).reshape(NS, nchunk, CHUNK)
    src_p = jnp.stack([src_t, src_t]).reshape(NW, nchunk, CHUNK)
    dst_p = jnp.tile(dst_t[None], (NC, 1, 1, 1)).reshape(NW, nchunk, CHUNK)

    n_acc = -(-(n + 1) // (NS * 8)) * (NS * 8)
    zeros = jnp.zeros((n_acc, half), jnp.float32)
    zeros16 = jnp.zeros((n_acc, DEGW), jnp.float32)
    ones = jnp.ones((CHUNK, DEGW), jnp.float32)

    agg_deg = _sc_aggregate_build(n_acc, half, nchunk, True, gw=d)
    agg = _sc_aggregate_build(n_acc, half, nchunk, False, gw=d)

    block_rows = 1000 if n % 1000 == 0 else 8

    def fsplit(h):
        return h.reshape(n, 2, half).transpose(1, 0, 2).reshape(2 * n, half)

    def chalves(c_flat):
        cp = c_flat.reshape(NC, n_acc, half)
        return cp[0, :n], cp[1, :n]

    c_flat, d_flat = agg_deg(x, src_p, dst_p, zeros, zeros16, ones)
    cl, cr = chalves(c_flat)
    dg = d_flat.reshape(NC, n_acc, DEGW)[0, :n]
    h = _tc_layer(x, cl, cr, dg, W0, b0, True, block_rows)

    (c_flat,) = agg(h, src_p, dst_p, zeros, zeros16, ones)
    cl, cr = chalves(c_flat)
    h = _tc_layer(h, cl, cr, dg, W1, b1, True, block_rows)

    (c_flat,) = agg(h, src_p, dst_p, zeros, zeros16, ones)
    cl, cr = chalves(c_flat)
    h = _tc_layer(h, cl, cr, dg, W2, b2, False, block_rows)
    return h


# speedup vs baseline: 1.6014x; 1.6014x over previous
import functools

import jax
import jax.numpy as jnp
from jax import lax
from jax.experimental import pallas as pl
from jax.experimental.pallas import tpu as pltpu
from jax.experimental.pallas import tpu_sc as plsc

NC = 2
NS = 16
NW = NC * NS
CHUNK = 128
GRP = 1
NBUF = 2
DEGW = 16


def _sc_aggregate_build(n_acc, half, nchunk, with_deg):
    rpt = n_acc // NS

    out_type = [jax.ShapeDtypeStruct((NC * n_acc, half), jnp.float32)]
    scratch = [
        pltpu.VMEM((nchunk, CHUNK), jnp.int32),
        pltpu.VMEM((nchunk, CHUNK), jnp.int32),
        pltpu.VMEM((NBUF, CHUNK, half), jnp.float32),
        pltpu.VMEM_SHARED((n_acc, half), jnp.float32),
    ] + [pltpu.SemaphoreType.DMA] * NBUF
    if with_deg:
        out_type.append(jax.ShapeDtypeStruct((NC * n_acc, DEGW), jnp.float32))
        scratch.append(pltpu.VMEM((CHUNK, DEGW), jnp.float32))
        scratch.append(pltpu.VMEM_SHARED((n_acc, DEGW), jnp.float32))

    mesh = plsc.VectorSubcoreMesh(core_axis_name="c", subcore_axis_name="s")

    def body(h_hbm, src_hbm, dst_hbm, zeros_hbm, zeros16_hbm, ones_hbm,
             *refs):
        if with_deg:
            (c_out, d_out, src_v, dst_v, rows_v, acc_s, *sems,
             ones_v, deg_s) = refs
        else:
            (c_out, src_v, dst_v, rows_v, acc_s, *sems) = refs
        gsem = sems
        cid = lax.axis_index("c")
        sid = lax.axis_index("s")
        wid = cid * NS + sid
        r0 = sid * rpt

        ngroups = nchunk // GRP

        def sidx(g):
            return src_v.at[g]

        def didx(g):
            return dst_v.at[g]

        pltpu.sync_copy(src_hbm.at[wid], src_v)
        for b in range(NBUF):
            pltpu.async_copy(h_hbm.at[sidx(b)], rows_v.at[b], gsem[b])

        pltpu.sync_copy(zeros_hbm.at[pl.ds(r0, rpt)], acc_s.at[pl.ds(r0, rpt)])
        if with_deg:
            pltpu.sync_copy(zeros16_hbm.at[pl.ds(r0, rpt)],
                            deg_s.at[pl.ds(r0, rpt)])
            pltpu.sync_copy(ones_hbm, ones_v)
        pltpu.sync_copy(dst_hbm.at[wid], dst_v)
        plsc.subcore_barrier()

        def step(s, carry):
            for b in range(NBUF):
                g = s * NBUF + b
                pltpu.make_async_copy(
                    h_hbm.at[sidx(g)], rows_v.at[b], gsem[b]).wait()
                pltpu.sync_copy(rows_v.at[b], acc_s.at[didx(g)], add=True)
                if with_deg:
                    pltpu.sync_copy(ones_v, deg_s.at[didx(g)], add=True)

                @pl.when(g + NBUF < ngroups)
                def _():
                    pltpu.async_copy(
                        h_hbm.at[sidx(g + NBUF)], rows_v.at[b], gsem[b])
            return carry

        lax.fori_loop(0, ngroups // NBUF, step, 0)
        plsc.subcore_barrier()

        base = cid * n_acc + r0
        pltpu.sync_copy(acc_s.at[pl.ds(r0, rpt)], c_out.at[pl.ds(base, rpt)])
        if with_deg:
            pltpu.sync_copy(deg_s.at[pl.ds(r0, rpt)],
                            d_out.at[pl.ds(base, rpt)])

    return pl.kernel(body, out_type=out_type, scratch_types=scratch,
                     mesh=mesh,
                     compiler_params=pltpu.CompilerParams(
                         use_tc_tiling_on_sc=False))


def _tc_layer_body(act, h_ref, cl_ref, cr_ref, dg_ref, w_ref, b_ref, o_ref):
    c = jnp.concatenate([cl_ref[...], cr_ref[...]], axis=1)
    deg = dg_ref[:, 0:1]
    c = c / jnp.maximum(deg, 1.0)
    h = h_ref[...]
    dh = h.shape[1]
    bundle = (jnp.dot(h, w_ref[:dh, :], preferred_element_type=jnp.float32)
              + jnp.dot(c, w_ref[dh:, :], preferred_element_type=jnp.float32)
              + b_ref[...])
    nrm = jnp.maximum(
        jnp.sqrt(jnp.sum(bundle * bundle, axis=1, keepdims=True)), 1e-12)
    bundle = bundle / nrm
    if act:
        bundle = jnp.maximum(bundle, 0.0)
    o_ref[...] = bundle


def _tc_layer(h, cl, cr, dg, w, b, act, block_rows):
    n, d = h.shape
    half = d // 2
    grid = (n // block_rows,)
    row_spec = pl.BlockSpec((block_rows, d), lambda i: (i, 0))
    half_spec = pl.BlockSpec((block_rows, half), lambda i: (i, 0))
    deg_spec = pl.BlockSpec((block_rows, DEGW), lambda i: (i, 0))
    return pl.pallas_call(
        functools.partial(_tc_layer_body, act),
        grid=grid,
        in_specs=[
            row_spec, half_spec, half_spec, deg_spec,
            pl.BlockSpec((2 * d, d), lambda i: (0, 0)),
            pl.BlockSpec((1, d), lambda i: (0, 0)),
        ],
        out_specs=row_spec,
        out_shape=jax.ShapeDtypeStruct((n, d), jnp.float32),
    )(h, cl, cr, dg, w, b.reshape(1, d))


def kernel(x, edge_index, W0, b0, W1, b1, W2, b2):
    n, d = x.shape
    e = edge_index.shape[1]
    half = d // 2

    per = -(-e // NS)
    q = GRP * NBUF
    nchunk = -(-(-(-per // CHUNK)) // q) * q
    pt = nchunk * CHUNK
    src = edge_index[0]
    dst = edge_index[1]
    pad_total = NS * pt - e
    src_t = jnp.concatenate(
        [src, jnp.zeros((pad_total,), jnp.int32)]).reshape(NS, nchunk, CHUNK)
    dst_t = jnp.concatenate(
        [dst, jnp.full((pad_total,), n, jnp.int32)]).reshape(NS, nchunk, CHUNK)
    src_p = jnp.stack([2 * src_t, 2 * src_t + 1]).reshape(NW, nchunk, CHUNK)
    dst_p = jnp.tile(dst_t[None], (NC, 1, 1, 1)).reshape(NW, nchunk, CHUNK)

    n_acc = -(-(n + 1) // (NS * 8)) * (NS * 8)
    zeros = jnp.zeros((n_acc, half), jnp.float32)
    zeros16 = jnp.zeros((n_acc, DEGW), jnp.float32)
    ones = jnp.ones((CHUNK, DEGW), jnp.float32)

    agg_deg = _sc_aggregate_build(n_acc, half, nchunk, True)
    agg = _sc_aggregate_build(n_acc, half, nchunk, False)

    block_rows = 1000 if n % 1000 == 0 else 8

    def fsplit(h):
        return h.reshape(2 * n, half)

    def chalves(c_flat):
        cp = c_flat.reshape(NC, n_acc, half)
        return cp[0, :n], cp[1, :n]

    c_flat, d_flat = agg_deg(fsplit(x), src_p, dst_p, zeros, zeros16, ones)
    cl, cr = chalves(c_flat)
    dg = d_flat.reshape(NC, n_acc, DEGW)[0, :n]
    h = _tc_layer(x, cl, cr, dg, W0, b0, True, block_rows)

    (c_flat,) = agg(fsplit(h), src_p, dst_p, zeros, zeros16, ones)
    cl, cr = chalves(c_flat)
    h = _tc_layer(h, cl, cr, dg, W1, b1, True, block_rows)

    (c_flat,) = agg(fsplit(h), src_p, dst_p, zeros, zeros16, ones)
    cl, cr = chalves(c_flat)
    h = _tc_layer(h, cl, cr, dg, W2, b2, False, block_rows)
    return h


# speedup vs baseline: 1.7619x; 1.1002x over previous
import functools

import jax
import jax.numpy as jnp
from jax import lax
from jax.experimental import pallas as pl
from jax.experimental.pallas import tpu as pltpu
from jax.experimental.pallas import tpu_sc as plsc

NC = 2
NS = 16
NW = NC * NS
CHUNK = 128
GRP = 1
NBUF = 2
DEGW = 16


def _sc_aggregate_build(n_acc, half, nchunk, with_deg):
    rpt = n_acc // NS

    out_type = [jax.ShapeDtypeStruct((NC * n_acc, half), jnp.float32)]
    scratch = [
        pltpu.VMEM((nchunk, CHUNK), jnp.int32),
        pltpu.VMEM((nchunk, CHUNK), jnp.int32),
        pltpu.VMEM((NBUF, CHUNK, half), jnp.float32),
        pltpu.VMEM_SHARED((n_acc, half), jnp.float32),
    ] + [pltpu.SemaphoreType.DMA] * NBUF
    if with_deg:
        out_type.append(jax.ShapeDtypeStruct((NC * n_acc, DEGW), jnp.float32))
        scratch.append(pltpu.VMEM((CHUNK, DEGW), jnp.float32))
        scratch.append(pltpu.VMEM_SHARED((n_acc, DEGW), jnp.float32))

    mesh = plsc.VectorSubcoreMesh(core_axis_name="c", subcore_axis_name="s")

    def body(h_hbm, src_hbm, dst_hbm, zeros_hbm, zeros16_hbm, ones_hbm,
             *refs):
        if with_deg:
            (c_out, d_out, src_v, dst_v, rows_v, acc_s, *sems,
             ones_v, deg_s) = refs
        else:
            (c_out, src_v, dst_v, rows_v, acc_s, *sems) = refs
        gsem = sems
        cid = lax.axis_index("c")
        sid = lax.axis_index("s")
        wid = cid * NS + sid
        r0 = sid * rpt

        ngroups = nchunk // GRP

        def sidx(g):
            return src_v.at[g]

        def didx(g):
            return dst_v.at[g]

        pltpu.sync_copy(src_hbm.at[wid], src_v)
        for b in range(NBUF):
            pltpu.async_copy(h_hbm.at[sidx(b)], rows_v.at[b], gsem[b])

        pltpu.sync_copy(zeros_hbm.at[pl.ds(r0, rpt)], acc_s.at[pl.ds(r0, rpt)])
        if with_deg:
            pltpu.sync_copy(zeros16_hbm.at[pl.ds(r0, rpt)],
                            deg_s.at[pl.ds(r0, rpt)])
            pltpu.sync_copy(ones_hbm, ones_v)
        pltpu.sync_copy(dst_hbm.at[wid], dst_v)
        plsc.subcore_barrier()

        def step(s, carry):
            for b in range(NBUF):
                g = s * NBUF + b
                pltpu.make_async_copy(
                    h_hbm.at[sidx(g)], rows_v.at[b], gsem[b]).wait()
                pltpu.sync_copy(rows_v.at[b], acc_s.at[didx(g)], add=True)
                if with_deg:
                    pltpu.sync_copy(ones_v, deg_s.at[didx(g)], add=True)

                @pl.when(g + NBUF < ngroups)
                def _():
                    pltpu.async_copy(
                        h_hbm.at[sidx(g + NBUF)], rows_v.at[b], gsem[b])
            return carry

        lax.fori_loop(0, ngroups // NBUF, step, 0)
        plsc.subcore_barrier()

        base = cid * n_acc + r0
        pltpu.sync_copy(acc_s.at[pl.ds(r0, rpt)], c_out.at[pl.ds(base, rpt)])
        if with_deg:
            pltpu.sync_copy(deg_s.at[pl.ds(r0, rpt)],
                            d_out.at[pl.ds(base, rpt)])

    return pl.kernel(body, out_type=out_type, scratch_types=scratch,
                     mesh=mesh,
                     compiler_params=pltpu.CompilerParams(
                         use_tc_tiling_on_sc=False))


def _tc_layer_body(act, h_ref, cl_ref, cr_ref, dg_ref, w_ref, b_ref, o_ref,
                   o2_ref):
    c = jnp.concatenate([cl_ref[...], cr_ref[...]], axis=1)
    deg = dg_ref[:, 0:1]
    c = c / jnp.maximum(deg, 1.0)
    h = h_ref[...]
    dh = h.shape[1]
    bundle = (jnp.dot(h, w_ref[:dh, :], preferred_element_type=jnp.float32)
              + jnp.dot(c, w_ref[dh:, :], preferred_element_type=jnp.float32)
              + b_ref[...])
    nrm = jnp.maximum(
        jnp.sqrt(jnp.sum(bundle * bundle, axis=1, keepdims=True)), 1e-12)
    bundle = bundle / nrm
    if act:
        bundle = jnp.maximum(bundle, 0.0)
    o_ref[...] = bundle
    dhalf = bundle.shape[1] // 2
    o2_ref[...] = jnp.stack([bundle[:, :dhalf], bundle[:, dhalf:]])


def _tc_layer(h, cl, cr, dg, w, b, act, block_rows):
    n, d = h.shape
    half = d // 2
    grid = (n // block_rows,)
    row_spec = pl.BlockSpec((block_rows, d), lambda i: (i, 0))
    half_spec = pl.BlockSpec((block_rows, half), lambda i: (i, 0))
    deg_spec = pl.BlockSpec((block_rows, DEGW), lambda i: (i, 0))
    return pl.pallas_call(
        functools.partial(_tc_layer_body, act),
        grid=grid,
        in_specs=[
            row_spec, half_spec, half_spec, deg_spec,
            pl.BlockSpec((2 * d, d), lambda i: (0, 0)),
            pl.BlockSpec((1, d), lambda i: (0, 0)),
        ],
        out_specs=[row_spec,
                   pl.BlockSpec((2, block_rows, half), lambda i: (0, i, 0))],
        out_shape=[jax.ShapeDtypeStruct((n, d), jnp.float32),
                   jax.ShapeDtypeStruct((2, n, half), jnp.float32)],
    )(h, cl, cr, dg, w, b.reshape(1, d))


def kernel(x, edge_index, W0, b0, W1, b1, W2, b2):
    n, d = x.shape
    e = edge_index.shape[1]
    half = d // 2

    per = -(-e // NS)
    q = GRP * NBUF
    nchunk = -(-(-(-per // CHUNK)) // q) * q
    pt = nchunk * CHUNK
    src = edge_index[0]
    dst = edge_index[1]
    pad_total = NS * pt - e
    src_t = jnp.concatenate(
        [src, jnp.zeros((pad_total,), jnp.int32)]).reshape(NS, nchunk, CHUNK)
    dst_t = jnp.concatenate(
        [dst, jnp.full((pad_total,), n, jnp.int32)]).reshape(NS, nchunk, CHUNK)
    src_p = jnp.stack([src_t, src_t + n]).reshape(NW, nchunk, CHUNK)
    dst_p = jnp.tile(dst_t[None], (NC, 1, 1, 1)).reshape(NW, nchunk, CHUNK)

    n_acc = -(-(n + 1) // (NS * 8)) * (NS * 8)
    zeros = jnp.zeros((n_acc, half), jnp.float32)
    zeros16 = jnp.zeros((n_acc, DEGW), jnp.float32)
    ones = jnp.ones((CHUNK, DEGW), jnp.float32)

    agg_deg = _sc_aggregate_build(n_acc, half, nchunk, True)
    agg = _sc_aggregate_build(n_acc, half, nchunk, False)

    block_rows = 1000 if n % 1000 == 0 else 8

    def fsplit(h):
        return h.reshape(n, 2, half).transpose(1, 0, 2).reshape(2 * n, half)

    def chalves(c_flat):
        cp = c_flat.reshape(NC, n_acc, half)
        return cp[0, :n], cp[1, :n]

    c_flat, d_flat = agg_deg(fsplit(x), src_p, dst_p, zeros, zeros16, ones)
    cl, cr = chalves(c_flat)
    dg = d_flat.reshape(NC, n_acc, DEGW)[0, :n]
    h, h2 = _tc_layer(x, cl, cr, dg, W0, b0, True, block_rows)

    (c_flat,) = agg(h2.reshape(2 * n, half), src_p, dst_p, zeros, zeros16,
                    ones)
    cl, cr = chalves(c_flat)
    h, h2 = _tc_layer(h, cl, cr, dg, W1, b1, True, block_rows)

    (c_flat,) = agg(h2.reshape(2 * n, half), src_p, dst_p, zeros, zeros16,
                    ones)
    cl, cr = chalves(c_flat)
    h, _ = _tc_layer(h, cl, cr, dg, W2, b2, False, block_rows)
    return h


# speedup vs baseline: 2.4007x; 1.3626x over previous
import functools

import jax
import jax.numpy as jnp
from jax import lax
from jax.experimental import pallas as pl
from jax.experimental.pallas import tpu as pltpu
from jax.experimental.pallas import tpu_sc as plsc

NC = 2
NS = 16
NW = NC * NS
CHUNK = 128
GRP = 1
NBUF = 2
DEGW = 16


def _sc_aggregate_build(n_acc, half, nchunk, with_deg):
    rpt = n_acc // NS

    out_type = [jax.ShapeDtypeStruct((NC * n_acc, half), jnp.bfloat16)]
    scratch = [
        pltpu.VMEM((nchunk, CHUNK), jnp.int32),
        pltpu.VMEM((nchunk, CHUNK), jnp.int32),
        pltpu.VMEM((NBUF, CHUNK, half), jnp.bfloat16),
        pltpu.VMEM_SHARED((n_acc, half), jnp.bfloat16),
    ] + [pltpu.SemaphoreType.DMA] * NBUF
    if with_deg:
        out_type.append(jax.ShapeDtypeStruct((NC * n_acc, DEGW), jnp.float32))
        scratch.append(pltpu.VMEM((CHUNK, DEGW), jnp.float32))
        scratch.append(pltpu.VMEM_SHARED((n_acc, DEGW), jnp.float32))

    mesh = plsc.VectorSubcoreMesh(core_axis_name="c", subcore_axis_name="s")

    def body(h_hbm, src_hbm, dst_hbm, zeros_hbm, zeros16_hbm, ones_hbm,
             *refs):
        if with_deg:
            (c_out, d_out, src_v, dst_v, rows_v, acc_s, *sems,
             ones_v, deg_s) = refs
        else:
            (c_out, src_v, dst_v, rows_v, acc_s, *sems) = refs
        gsem = sems
        cid = lax.axis_index("c")
        sid = lax.axis_index("s")
        wid = cid * NS + sid
        r0 = sid * rpt

        ngroups = nchunk // GRP

        def sidx(g):
            return src_v.at[g]

        def didx(g):
            return dst_v.at[g]

        pltpu.sync_copy(src_hbm.at[wid], src_v)
        for b in range(NBUF):
            pltpu.async_copy(h_hbm.at[sidx(b)], rows_v.at[b], gsem[b])

        pltpu.sync_copy(zeros_hbm.at[pl.ds(r0, rpt)], acc_s.at[pl.ds(r0, rpt)])
        if with_deg:
            pltpu.sync_copy(zeros16_hbm.at[pl.ds(r0, rpt)],
                            deg_s.at[pl.ds(r0, rpt)])
            pltpu.sync_copy(ones_hbm, ones_v)
        pltpu.sync_copy(dst_hbm.at[wid], dst_v)
        plsc.subcore_barrier()

        def step(s, carry):
            for b in range(NBUF):
                g = s * NBUF + b
                pltpu.make_async_copy(
                    h_hbm.at[sidx(g)], rows_v.at[b], gsem[b]).wait()
                pltpu.sync_copy(rows_v.at[b], acc_s.at[didx(g)], add=True)
                if with_deg:
                    pltpu.sync_copy(ones_v, deg_s.at[didx(g)], add=True)

                @pl.when(g + NBUF < ngroups)
                def _():
                    pltpu.async_copy(
                        h_hbm.at[sidx(g + NBUF)], rows_v.at[b], gsem[b])
            return carry

        lax.fori_loop(0, ngroups // NBUF, step, 0)
        plsc.subcore_barrier()

        base = cid * n_acc + r0
        pltpu.sync_copy(acc_s.at[pl.ds(r0, rpt)], c_out.at[pl.ds(base, rpt)])
        if with_deg:
            pltpu.sync_copy(deg_s.at[pl.ds(r0, rpt)],
                            d_out.at[pl.ds(base, rpt)])

    return pl.kernel(body, out_type=out_type, scratch_types=scratch,
                     mesh=mesh,
                     compiler_params=pltpu.CompilerParams(
                         use_tc_tiling_on_sc=False))


def _tc_layer_body(act, h_ref, cl_ref, cr_ref, dg_ref, w_ref, b_ref, o_ref):
    c = jnp.concatenate([cl_ref[...], cr_ref[...]],
                        axis=1).astype(jnp.float32)
    deg = dg_ref[:, 0:1]
    c = c / jnp.maximum(deg, 1.0)
    h = h_ref[...]
    dh = h.shape[1]
    bundle = (jnp.dot(h, w_ref[:dh, :], preferred_element_type=jnp.float32)
              + jnp.dot(c, w_ref[dh:, :], preferred_element_type=jnp.float32)
              + b_ref[...])
    nrm = jnp.maximum(
        jnp.sqrt(jnp.sum(bundle * bundle, axis=1, keepdims=True)), 1e-12)
    bundle = bundle / nrm
    if act:
        bundle = jnp.maximum(bundle, 0.0)
    o_ref[...] = bundle


def _tc_layer(h, cl, cr, dg, w, b, act, block_rows):
    n, d = h.shape
    half = d // 2
    grid = (n // block_rows,)
    row_spec = pl.BlockSpec((block_rows, d), lambda i: (i, 0))
    half_spec = pl.BlockSpec((block_rows, half), lambda i: (i, 0))
    deg_spec = pl.BlockSpec((block_rows, DEGW), lambda i: (i, 0))
    return pl.pallas_call(
        functools.partial(_tc_layer_body, act),
        grid=grid,
        in_specs=[
            row_spec, half_spec, half_spec, deg_spec,
            pl.BlockSpec((2 * d, d), lambda i: (0, 0)),
            pl.BlockSpec((1, d), lambda i: (0, 0)),
        ],
        out_specs=row_spec,
        out_shape=jax.ShapeDtypeStruct((n, d), jnp.float32),
    )(h, cl, cr, dg, w, b.reshape(1, d))


def kernel(x, edge_index, W0, b0, W1, b1, W2, b2):
    n, d = x.shape
    e = edge_index.shape[1]
    half = d // 2

    per = -(-e // NS)
    q = GRP * NBUF
    nchunk = -(-(-(-per // CHUNK)) // q) * q
    pt = nchunk * CHUNK
    src = edge_index[0]
    dst = edge_index[1]
    pad_total = NS * pt - e
    src_t = jnp.concatenate(
        [src, jnp.zeros((pad_total,), jnp.int32)]).reshape(NS, nchunk, CHUNK)
    dst_t = jnp.concatenate(
        [dst, jnp.full((pad_total,), n, jnp.int32)]).reshape(NS, nchunk, CHUNK)
    src_p = jnp.stack([src_t, src_t + n]).reshape(NW, nchunk, CHUNK)
    dst_p = jnp.tile(dst_t[None], (NC, 1, 1, 1)).reshape(NW, nchunk, CHUNK)

    n_acc = -(-(n + 1) // (NS * 8)) * (NS * 8)
    zeros = jnp.zeros((n_acc, half), jnp.bfloat16)
    zeros16 = jnp.zeros((n_acc, DEGW), jnp.float32)
    ones = jnp.ones((CHUNK, DEGW), jnp.float32)

    agg_deg = _sc_aggregate_build(n_acc, half, nchunk, True)
    agg = _sc_aggregate_build(n_acc, half, nchunk, False)

    block_rows = 2000 if n % 2000 == 0 else 16

    def fsplit(h):
        return h.reshape(n, 2, half).transpose(1, 0, 2).reshape(
            2 * n, half).astype(jnp.bfloat16)

    def chalves(c_flat):
        cp = c_flat.reshape(NC, n_acc, half)
        return cp[0, :n], cp[1, :n]

    c_flat, d_flat = agg_deg(fsplit(x), src_p, dst_p, zeros, zeros16, ones)
    cl, cr = chalves(c_flat)
    dg = d_flat.reshape(NC, n_acc, DEGW)[0, :n]
    h = _tc_layer(x, cl, cr, dg, W0, b0, True, block_rows)

    (c_flat,) = agg(fsplit(h), src_p, dst_p, zeros, zeros16, ones)
    cl, cr = chalves(c_flat)
    h = _tc_layer(h, cl, cr, dg, W1, b1, True, block_rows)

    (c_flat,) = agg(fsplit(h), src_p, dst_p, zeros, zeros16, ones)
    cl, cr = chalves(c_flat)
    h = _tc_layer(h, cl, cr, dg, W2, b2, False, block_rows)
    return h
